# Initial kernel scaffold; baseline (speedup 1.0000x reference)
import jax
import jax.numpy as jnp
from jax.experimental import pallas as pl


def kernel(des1, det1, qlt1, des2, det2, qlt2, aflow):
    raise NotImplementedError("write your pallas kernel here")


# speedup vs baseline: 1.2328x; 1.2328x over previous
import functools

import jax
import jax.numpy as jnp
from jax import lax
from jax.experimental import pallas as pl
from jax.experimental.pallas import tpu as pltpu
from jax.experimental.pallas import tpu_sc as plsc

H = W = 512
HW = H * W
D = 128
CELL = 16
BORDER = 16
NC = 30
N = NC * NC
NP = 1024
POS_R = 2

_OFFS = [(i, j) for i in range(-POS_R, POS_R + 1)
         for j in range(-POS_R, POS_R + 1) if i * i + j * j <= POS_R ** 2]
K = len(_OFFS)

NTILES = 32
SPT = NP // NTILES
HSPT = SPT // 2


def _sc_mesh():
    return plsc.VectorSubcoreMesh(core_axis_name="c", subcore_axis_name="s",
                                  num_cores=2, num_subcores=16)


def _argmax_kernel(d1_ref, d2_ref, p1_ref, pd_ref, drow_ref, dcol_ref):
    lane = lax.broadcasted_iota(jnp.int32, (1, NP), 1)
    ci = lane // NC
    cj = lane % NC

    def cell_argmax(x):
        m = jnp.max(x, axis=0, keepdims=True)
        ii = lax.broadcasted_iota(jnp.int32, x.shape, 0)
        return jnp.min(jnp.where(x >= m, ii, 256), axis=0, keepdims=True)

    i1 = cell_argmax(d1_ref[...])
    i2 = cell_argmax(d2_ref[...])

    def coords(i):
        ri = i // CELL
        rj = i % CELL
        sx = jnp.clip(BORDER + cj * CELL + rj, 0, W - 1)
        sy = jnp.clip(BORDER + ci * CELL + ri, 0, H - 1)
        return sx, sy

    sx1, sy1 = coords(i1)
    sx2, sy2 = coords(i2)
    p1_ref[...] = sx1 * W + sy1
    pd_ref[...] = sx2 * W + sy2
    drow_ref[...] = sy2
    dcol_ref[...] = sx2


def _run_argmax(d1r, d2r):
    return pl.pallas_call(
        _argmax_kernel,
        out_shape=[jax.ShapeDtypeStruct((1, NP), jnp.int32)] * 4,
    )(d1r, d2r)


def _sc_flow_body(p1_hbm, aflowf, qlt1f, qlt2f,
                  q1_o, xx_o, yy_o, mk_o, qc_o,
                  p1v, axv, ayv, q1v, xy2xv, xy2yv, maskv, qcv, sem):
    wid = lax.axis_index("s") * 2 + lax.axis_index("c")
    base = wid * SPT

    pltpu.sync_copy(p1_hbm.at[pl.ds(base, SPT)], p1v)

    cps = []
    for c in range(SPT // 16):
        sl = pl.ds(16 * c, 16)
        pch = p1v[sl]
        cps.append(pltpu.async_copy(aflowf.at[pch], axv.at[sl], sem))
        cps.append(pltpu.async_copy(aflowf.at[pch + HW], ayv.at[sl], sem))
        cps.append(pltpu.async_copy(qlt1f.at[pch], q1v.at[sl], sem))
    for cp in cps:
        cp.wait()

    qcps = []
    for c in range(SPT // 16):
        sl = pl.ds(16 * c, 16)
        xx = (axv[sl] + 0.5).astype(jnp.int32)
        yy = (ayv[sl] + 0.5).astype(jnp.int32)
        inb = (xx >= 0) & (xx < W) & (yy >= 0) & (yy < H)
        xy2xv[sl] = xx
        xy2yv[sl] = yy
        maskv[sl] = jnp.where(inb, 1, 0)
        for k, (oi, oj) in enumerate(_OFFS):
            nx = jnp.clip(xx + oi, 0, W - 1)
            ny = jnp.clip(yy + oj, 0, H - 1)
            pn = ny * W + nx
            qcps.append(pltpu.async_copy(qlt2f.at[pn], qcv.at[k, sl], sem))
    for cp in qcps:
        cp.wait()

    pltpu.sync_copy(q1v, q1_o.at[pl.ds(base, SPT)])
    pltpu.sync_copy(xy2xv, xx_o.at[pl.ds(base, SPT)])
    pltpu.sync_copy(xy2yv, yy_o.at[pl.ds(base, SPT)])
    pltpu.sync_copy(maskv, mk_o.at[pl.ds(base, SPT)])
    pltpu.sync_copy(qcv, qc_o.at[wid])


@functools.cache
def _build_sc_flow():
    return pl.kernel(
        _sc_flow_body,
        out_type=[
            jax.ShapeDtypeStruct((NP,), jnp.float32),
            jax.ShapeDtypeStruct((NP,), jnp.int32),
            jax.ShapeDtypeStruct((NP,), jnp.int32),
            jax.ShapeDtypeStruct((NP,), jnp.int32),
            jax.ShapeDtypeStruct((NTILES, 16, SPT), jnp.float32),
        ],
        mesh=_sc_mesh(),
        scratch_types=[
            pltpu.VMEM((SPT,), jnp.int32),
            pltpu.VMEM((SPT,), jnp.float32),
            pltpu.VMEM((SPT,), jnp.float32),
            pltpu.VMEM((SPT,), jnp.float32),
            pltpu.VMEM((SPT,), jnp.int32),
            pltpu.VMEM((SPT,), jnp.int32),
            pltpu.VMEM((SPT,), jnp.int32),
            pltpu.VMEM((16, SPT), jnp.float32),
            pltpu.SemaphoreType.DMA,
        ],
    )


def _lists_kernel(p1c_ref, pdc_ref, xxc_ref, yyc_ref,
                  idx1_ref, idxd_ref, idxn_ref):
    di = lax.broadcasted_iota(jnp.int32, (NP, D), 1) * HW
    idx1_ref[...] = p1c_ref[...] + di
    idxd_ref[...] = pdc_ref[...] + di
    xx = xxc_ref[...]
    yy = yyc_ref[...]
    for k, (oi, oj) in enumerate(_OFFS):
        nx = jnp.clip(xx + oi, 0, W - 1)
        ny = jnp.clip(yy + oj, 0, H - 1)
        idxn_ref[k] = (ny * W + nx) + di


def _run_lists(p1c, pdc, xxc, yyc):
    return pl.pallas_call(
        _lists_kernel,
        out_shape=[
            jax.ShapeDtypeStruct((NP, D), jnp.int32),
            jax.ShapeDtypeStruct((NP, D), jnp.int32),
            jax.ShapeDtypeStruct((K, NP, D), jnp.int32),
        ],
    )(p1c, pdc, xxc, yyc)


def _sc_desc_body(idx1_hbm, idxd_hbm, idxn_hbm, des1f, des2f,
                  sdes_o, distr_o, neigh_o,
                  idxsv, sdv, idxnv, ngv, sem):
    wid = lax.axis_index("s") * 2 + lax.axis_index("c")
    base = wid * SPT

    for h in range(SPT // HSPT):
        hb = base + h * HSPT
        pltpu.sync_copy(idx1_hbm.at[pl.ds(hb, HSPT)], idxsv.at[0])
        pltpu.sync_copy(idxd_hbm.at[pl.ds(hb, HSPT)], idxsv.at[1])
        cps = []
        for i in range(HSPT):
            cps.append(pltpu.async_copy(
                des1f.at[idxsv.at[0, i]], sdv.at[0, i], sem))
            cps.append(pltpu.async_copy(
                des2f.at[idxsv.at[1, i]], sdv.at[1, i], sem))
        for cp in cps:
            cp.wait()
        pltpu.sync_copy(sdv.at[0], sdes_o.at[pl.ds(hb, HSPT)])
        pltpu.sync_copy(sdv.at[1], distr_o.at[pl.ds(hb, HSPT)])

        pltpu.sync_copy(idxn_hbm.at[:, pl.ds(hb, HSPT)], idxnv)
        cps = []
        for k in range(K):
            for i in range(HSPT):
                cps.append(pltpu.async_copy(
                    des2f.at[idxnv.at[k, i]], ngv.at[k, i], sem))
        for cp in cps:
            cp.wait()
        pltpu.sync_copy(ngv, neigh_o.at[:, pl.ds(hb, HSPT)])


@functools.cache
def _build_sc_desc():
    return pl.kernel(
        _sc_desc_body,
        out_type=[
            jax.ShapeDtypeStruct((NP, D), jnp.float32),
            jax.ShapeDtypeStruct((NP, D), jnp.float32),
            jax.ShapeDtypeStruct((K, NP, D), jnp.float32),
        ],
        mesh=_sc_mesh(),
        scratch_types=[
            pltpu.VMEM((2, HSPT, D), jnp.int32),
            pltpu.VMEM((2, HSPT, D), jnp.float32),
            pltpu.VMEM((K, HSPT, D), jnp.int32),
            pltpu.VMEM((K, HSPT, D), jnp.float32),
            pltpu.SemaphoreType.DMA,
        ],
    )


def _score_kernel(sdes_ref, neigh_ref, distr_ref, qc_ref, q1c_ref,
                  xxc_ref, yyc_ref, drow_ref, dcol_ref,
                  mm_ref, mx_ref, qlt_ref):
    sdes = sdes_ref[...]
    mx = jnp.full((NP, 1), -jnp.inf, jnp.float32)
    qsel = jnp.zeros((NP, 1), jnp.float32)
    for k in range(K):
        s = jnp.sum(sdes * neigh_ref[k], axis=-1, keepdims=True)
        better = s > mx
        mx = jnp.where(better, s, mx)
        qsel = jnp.where(better, qc_ref[:, k:k + 1], qsel)
    mx_ref[...] = mx
    qlt_ref[...] = (q1c_ref[...] + qsel) * 0.5

    mm = lax.dot_general(sdes, distr_ref[...],
                         (((1,), (1,)), ((), ())),
                         preferred_element_type=jnp.float32)
    dx = drow_ref[...] - xxc_ref[...]
    dy = dcol_ref[...] - yyc_ref[...]
    dis2 = dx * dx + dy * dy
    mm_ref[...] = jnp.where(dis2 < POS_R ** 2, 0.0, mm)


def _run_score(sdes, neigh, distr, qc, q1c, xxc, yyc, drowr, dcolr):
    return pl.pallas_call(
        _score_kernel,
        out_shape=[
            jax.ShapeDtypeStruct((NP, NP), jnp.float32),
            jax.ShapeDtypeStruct((NP, 1), jnp.float32),
            jax.ShapeDtypeStruct((NP, 1), jnp.float32),
        ],
    )(sdes, neigh, distr, qc, q1c, xxc, yyc, drowr, dcolr)


def _unshuffle(det):
    c = det[0, 0, BORDER:H - BORDER, BORDER:W - BORDER]
    c = c.reshape(NC, CELL, NC, CELL).transpose(1, 3, 0, 2).reshape(CELL * CELL, N)
    return jnp.pad(c, ((0, 0), (0, NP - N)), constant_values=-1.0)


def kernel(des1, det1, qlt1, des2, det2, qlt2, aflow):
    d1r = _unshuffle(det1)
    d2r = _unshuffle(det2)
    p1, pd, drow, dcol = _run_argmax(d1r, d2r)

    q1, xx, yy, mk, qc = _build_sc_flow()(
        p1.reshape(NP), aflow.reshape(-1), qlt1.reshape(-1), qlt2.reshape(-1))

    idx1, idxd, idxn = _run_lists(
        p1.reshape(NP, 1), pd.reshape(NP, 1),
        xx.reshape(NP, 1), yy.reshape(NP, 1))

    sdes, distr, neigh = _build_sc_desc()(
        idx1, idxd, idxn, des1.reshape(-1), des2.reshape(-1))

    qcn = qc.transpose(0, 2, 1).reshape(NP, 16)
    mm, mxc, qltc = _run_score(
        sdes, neigh, distr, qcn, q1.reshape(NP, 1),
        xx.reshape(NP, 1), yy.reshape(NP, 1),
        drow.reshape(1, NP), dcol.reshape(1, NP))

    scores = jnp.concatenate([mxc[:N], mm[:N, :N]], axis=1)
    labels = jnp.zeros((N, N + 1), dtype=bool).at[:, :1].set(True)
    mask = (mk[:N] != 0).reshape(1, N)
    qlt = qltc[:N]
    return scores, labels, mask, qlt


# speedup vs baseline: 1.7352x; 1.4075x over previous
import functools

import jax
import jax.numpy as jnp
from jax import lax
from jax.experimental import pallas as pl
from jax.experimental.pallas import tpu as pltpu
from jax.experimental.pallas import tpu_sc as plsc

H = W = 512
HW = H * W
D = 128
CELL = 16
BORDER = 16
NC = 30
N = NC * NC
NP = 1024
POS_R = 2

_OFFS = [(i, j) for i in range(-POS_R, POS_R + 1)
         for j in range(-POS_R, POS_R + 1) if i * i + j * j <= POS_R ** 2]
K = len(_OFFS)

NTILES = 32
SPT = NP // NTILES


def _sc_mesh():
    return plsc.VectorSubcoreMesh(core_axis_name="c", subcore_axis_name="s",
                                  num_cores=2, num_subcores=16)


def _argmax_kernel(d1_ref, d2_ref, p1_ref, pd_ref, drow_ref, dcol_ref):
    lane = lax.broadcasted_iota(jnp.int32, (1, NP), 1)
    ci = lane // NC
    cj = lane % NC

    def cell_argmax(x):
        m = jnp.max(x, axis=0, keepdims=True)
        ii = lax.broadcasted_iota(jnp.int32, x.shape, 0)
        return jnp.min(jnp.where(x >= m, ii, 256), axis=0, keepdims=True)

    i1 = cell_argmax(d1_ref[...])
    i2 = cell_argmax(d2_ref[...])

    def coords(i):
        ri = i // CELL
        rj = i % CELL
        sx = jnp.clip(BORDER + cj * CELL + rj, 0, W - 1)
        sy = jnp.clip(BORDER + ci * CELL + ri, 0, H - 1)
        return sx, sy

    sx1, sy1 = coords(i1)
    sx2, sy2 = coords(i2)
    p1_ref[...] = sx1 * W + sy1
    pd_ref[...] = sx2 * W + sy2
    drow_ref[...] = sy2
    dcol_ref[...] = sx2


def _run_argmax(d1r, d2r):
    return pl.pallas_call(
        _argmax_kernel,
        out_shape=[jax.ShapeDtypeStruct((1, NP), jnp.int32)] * 4,
    )(d1r, d2r)


def _sc_gather_body(p1_hbm, pd_hbm, des1t, des2t, aflowf, qlt1f, qlt2f,
                    sdes_o, distr_o, neigh_o, qc_o, q1_o, xx_o, yy_o, mk_o,
                    p1v, pdv, axv, ayv, q1v, xy2xv, xy2yv, maskv, qcv,
                    sdv, dsv, ngv, sem, semd):
    wid = lax.axis_index("s") * 2 + lax.axis_index("c")
    base = wid * SPT

    pltpu.sync_copy(p1_hbm.at[pl.ds(base, SPT)], p1v)
    pltpu.sync_copy(pd_hbm.at[pl.ds(base, SPT)], pdv)

    dcps = []
    cps = []
    for c in range(SPT // 16):
        sl = pl.ds(16 * c, 16)
        pch = p1v[sl]
        dcps.append(pltpu.async_copy(des1t.at[pch], sdv.at[sl], semd))
        dcps.append(pltpu.async_copy(des2t.at[pdv[sl]], dsv.at[sl], semd))
        cps.append(pltpu.async_copy(aflowf.at[pch], axv.at[sl], sem))
        cps.append(pltpu.async_copy(aflowf.at[pch + HW], ayv.at[sl], sem))
        cps.append(pltpu.async_copy(qlt1f.at[pch], q1v.at[sl], sem))
    for cp in cps:
        cp.wait()

    qcps = []
    for c in range(SPT // 16):
        sl = pl.ds(16 * c, 16)
        xx = (axv[sl] + 0.5).astype(jnp.int32)
        yy = (ayv[sl] + 0.5).astype(jnp.int32)
        inb = (xx >= 0) & (xx < W) & (yy >= 0) & (yy < H)
        xy2xv[sl] = xx
        xy2yv[sl] = yy
        maskv[sl] = jnp.where(inb, 1, 0)
        for k, (oi, oj) in enumerate(_OFFS):
            nx = jnp.clip(xx + oi, 0, W - 1)
            ny = jnp.clip(yy + oj, 0, H - 1)
            pn = ny * W + nx
            qcps.append(pltpu.async_copy(qlt2f.at[pn], qcv.at[k, sl], sem))
            dcps.append(pltpu.async_copy(des2t.at[pn], ngv.at[k, sl], semd))
    for cp in qcps:
        cp.wait()

    pltpu.sync_copy(q1v, q1_o.at[pl.ds(base, SPT)])
    pltpu.sync_copy(xy2xv, xx_o.at[pl.ds(base, SPT)])
    pltpu.sync_copy(xy2yv, yy_o.at[pl.ds(base, SPT)])
    pltpu.sync_copy(maskv, mk_o.at[pl.ds(base, SPT)])
    pltpu.sync_copy(qcv, qc_o.at[wid])

    for cp in dcps:
        cp.wait()
    pltpu.sync_copy(sdv, sdes_o.at[pl.ds(base, SPT)])
    pltpu.sync_copy(dsv, distr_o.at[pl.ds(base, SPT)])
    pltpu.sync_copy(ngv, neigh_o.at[:, pl.ds(base, SPT)])


@functools.cache
def _build_sc_gather():
    return pl.kernel(
        _sc_gather_body,
        out_type=[
            jax.ShapeDtypeStruct((NP, D), jnp.float32),
            jax.ShapeDtypeStruct((NP, D), jnp.float32),
            jax.ShapeDtypeStruct((K, NP, D), jnp.float32),
            jax.ShapeDtypeStruct((NTILES, 16, SPT), jnp.float32),
            jax.ShapeDtypeStruct((NP,), jnp.float32),
            jax.ShapeDtypeStruct((NP,), jnp.int32),
            jax.ShapeDtypeStruct((NP,), jnp.int32),
            jax.ShapeDtypeStruct((NP,), jnp.int32),
        ],
        mesh=_sc_mesh(),
        scratch_types=[
            pltpu.VMEM((SPT,), jnp.int32),
            pltpu.VMEM((SPT,), jnp.int32),
            pltpu.VMEM((SPT,), jnp.float32),
            pltpu.VMEM((SPT,), jnp.float32),
            pltpu.VMEM((SPT,), jnp.float32),
            pltpu.VMEM((SPT,), jnp.int32),
            pltpu.VMEM((SPT,), jnp.int32),
            pltpu.VMEM((SPT,), jnp.int32),
            pltpu.VMEM((16, SPT), jnp.float32),
            pltpu.VMEM((SPT, D), jnp.float32),
            pltpu.VMEM((SPT, D), jnp.float32),
            pltpu.VMEM((K, SPT, D), jnp.float32),
            pltpu.SemaphoreType.DMA,
            pltpu.SemaphoreType.DMA,
        ],
    )


def _score_kernel(sdes_ref, neigh_ref, distr_ref, qc_ref, q1c_ref,
                  xxc_ref, yyc_ref, drow_ref, dcol_ref,
                  mm_ref, mx_ref, qlt_ref):
    sdes = sdes_ref[...]
    mx = jnp.full((NP, 1), -jnp.inf, jnp.float32)
    qsel = jnp.zeros((NP, 1), jnp.float32)
    for k in range(K):
        s = jnp.sum(sdes * neigh_ref[k], axis=-1, keepdims=True)
        better = s > mx
        mx = jnp.where(better, s, mx)
        qsel = jnp.where(better, qc_ref[:, k:k + 1], qsel)
    mx_ref[...] = mx
    qlt_ref[...] = (q1c_ref[...] + qsel) * 0.5

    mm = lax.dot_general(sdes, distr_ref[...],
                         (((1,), (1,)), ((), ())),
                         preferred_element_type=jnp.float32)
    dx = drow_ref[...] - xxc_ref[...]
    dy = dcol_ref[...] - yyc_ref[...]
    dis2 = dx * dx + dy * dy
    mm_ref[...] = jnp.where(dis2 < POS_R ** 2, 0.0, mm)


def _run_score(sdes, neigh, distr, qc, q1c, xxc, yyc, drowr, dcolr):
    return pl.pallas_call(
        _score_kernel,
        out_shape=[
            jax.ShapeDtypeStruct((NP, NP), jnp.float32),
            jax.ShapeDtypeStruct((NP, 1), jnp.float32),
            jax.ShapeDtypeStruct((NP, 1), jnp.float32),
        ],
    )(sdes, neigh, distr, qc, q1c, xxc, yyc, drowr, dcolr)


def _unshuffle(det):
    c = det[0, 0, BORDER:H - BORDER, BORDER:W - BORDER]
    c = c.reshape(NC, CELL, NC, CELL).transpose(1, 3, 0, 2).reshape(CELL * CELL, N)
    return jnp.pad(c, ((0, 0), (0, NP - N)), constant_values=-1.0)


def kernel(des1, det1, qlt1, des2, det2, qlt2, aflow):
    d1r = _unshuffle(det1)
    d2r = _unshuffle(det2)
    p1, pd, drow, dcol = _run_argmax(d1r, d2r)

    des1t = des1.reshape(D, HW).T
    des2t = des2.reshape(D, HW).T

    sdes, distr, neigh, qc, q1, xx, yy, mk = _build_sc_gather()(
        p1.reshape(NP), pd.reshape(NP), des1t, des2t,
        aflow.reshape(-1), qlt1.reshape(-1), qlt2.reshape(-1))

    qcn = qc.transpose(0, 2, 1).reshape(NP, 16)
    mm, mxc, qltc = _run_score(
        sdes, neigh, distr, qcn, q1.reshape(NP, 1),
        xx.reshape(NP, 1), yy.reshape(NP, 1),
        drow.reshape(1, NP), dcol.reshape(1, NP))

    scores = jnp.concatenate([mxc[:N], mm[:N, :N]], axis=1)
    labels = jnp.zeros((N, N + 1), dtype=bool).at[:, :1].set(True)
    mask = (mk[:N] != 0).reshape(1, N)
    qlt = qltc[:N]
    return scores, labels, mask, qlt


# speedup vs baseline: 2.0680x; 1.1917x over previous
import functools

import jax
import jax.numpy as jnp
from jax import lax
from jax.experimental import pallas as pl
from jax.experimental.pallas import tpu as pltpu
from jax.experimental.pallas import tpu_sc as plsc

H = W = 512
HW = H * W
D = 128
CELL = 16
BORDER = 16
NC = 30
N = NC * NC
NP = 1024
POS_R = 2

_OFFS = [(i, j) for i in range(-POS_R, POS_R + 1)
         for j in range(-POS_R, POS_R + 1) if i * i + j * j <= POS_R ** 2]
K = len(_OFFS)

NTILES = 32
SPT = NP // NTILES


def _sc_mesh():
    return plsc.VectorSubcoreMesh(core_axis_name="c", subcore_axis_name="s",
                                  num_cores=2, num_subcores=16)


def _argmax_kernel(d1_ref, d2_ref, p1_ref, pd_ref, drow_ref, dcol_ref,
                   tg1_ref):
    lane = lax.broadcasted_iota(jnp.int32, (1, NP), 1)
    ci = lane // NC
    cj = lane % NC

    def cell_argmax(x):
        m = jnp.max(x, axis=0, keepdims=True)
        ii = lax.broadcasted_iota(jnp.int32, x.shape, 0)
        return jnp.min(jnp.where(x >= m, ii, 256), axis=0, keepdims=True)

    i1 = cell_argmax(d1_ref[...])
    i2 = cell_argmax(d2_ref[...])

    def coords(i):
        ri = i // CELL
        rj = i % CELL
        sx = jnp.clip(BORDER + cj * CELL + rj, 0, W - 1)
        sy = jnp.clip(BORDER + ci * CELL + ri, 0, H - 1)
        return sx, sy

    sx1, sy1 = coords(i1)
    sx2, sy2 = coords(i2)
    p1_ref[...] = sx1 * W + sy1
    pd_ref[...] = sx2 * W + sy2
    drow_ref[...] = sy2
    dcol_ref[...] = sx2
    tg1_ref[...] = ((sx1 - BORDER) % CELL) * W + sy1


def _run_argmax(d1r, d2r):
    return pl.pallas_call(
        _argmax_kernel,
        out_shape=[jax.ShapeDtypeStruct((1, NP), jnp.int32)] * 5,
    )(d1r, d2r)


def _gather1_kernel(des1_ref, tgt_ref, out_ref):
    slab = des1_ref[...].reshape(D, CELL * W)
    t = tgt_ref[...].reshape(32, 1)
    onehot = (lax.broadcasted_iota(jnp.int32, (32, CELL * W), 1) == t
              ).astype(jnp.float32)
    out_ref[...] = lax.dot_general(
        onehot, slab, (((1,), (1,)), ((), ())),
        preferred_element_type=jnp.float32)[None]


def _run_gather1(des1_3d, tgt3):
    return pl.pallas_call(
        _gather1_kernel,
        grid=(NC,),
        in_specs=[
            pl.BlockSpec((D, CELL, W), lambda i: (0, i + 1, 0)),
            pl.BlockSpec((1, 1, 32), lambda i: (i, 0, 0)),
        ],
        out_specs=pl.BlockSpec((1, 32, D), lambda i: (i, 0, 0)),
        out_shape=jax.ShapeDtypeStruct((NC, 32, D), jnp.float32),
    )(des1_3d, tgt3)


def _sc_gather_body(p1_hbm, pd_hbm, des2t, aflowf, qlt1f, qlt2f,
                    distr_o, neigh_o, qc_o, q1_o, xx_o, yy_o, mk_o,
                    p1v, pdv, axv, ayv, q1v, xy2xv, xy2yv, maskv, qcv,
                    dsv, ngv, sem, semd):
    wid = lax.axis_index("s") * 2 + lax.axis_index("c")
    base = wid * SPT

    pltpu.sync_copy(p1_hbm.at[pl.ds(base, SPT)], p1v)
    pltpu.sync_copy(pd_hbm.at[pl.ds(base, SPT)], pdv)

    dcps = []
    cps = []
    for c in range(SPT // 16):
        sl = pl.ds(16 * c, 16)
        pch = p1v[sl]
        dcps.append(pltpu.async_copy(des2t.at[pdv[sl]], dsv.at[sl], semd))
        cps.append(pltpu.async_copy(aflowf.at[pch], axv.at[sl], sem))
        cps.append(pltpu.async_copy(aflowf.at[pch + HW], ayv.at[sl], sem))
        cps.append(pltpu.async_copy(qlt1f.at[pch], q1v.at[sl], sem))
    for cp in cps:
        cp.wait()

    qcps = []
    for c in range(SPT // 16):
        sl = pl.ds(16 * c, 16)
        xx = (axv[sl] + 0.5).astype(jnp.int32)
        yy = (ayv[sl] + 0.5).astype(jnp.int32)
        inb = (xx >= 0) & (xx < W) & (yy >= 0) & (yy < H)
        xy2xv[sl] = xx
        xy2yv[sl] = yy
        maskv[sl] = jnp.where(inb, 1, 0)
        for k, (oi, oj) in enumerate(_OFFS):
            nx = jnp.clip(xx + oi, 0, W - 1)
            ny = jnp.clip(yy + oj, 0, H - 1)
            pn = ny * W + nx
            qcps.append(pltpu.async_copy(qlt2f.at[pn], qcv.at[k, sl], sem))
            dcps.append(pltpu.async_copy(des2t.at[pn], ngv.at[k, sl], semd))
    for cp in qcps:
        cp.wait()

    pltpu.sync_copy(q1v, q1_o.at[pl.ds(base, SPT)])
    pltpu.sync_copy(xy2xv, xx_o.at[pl.ds(base, SPT)])
    pltpu.sync_copy(xy2yv, yy_o.at[pl.ds(base, SPT)])
    pltpu.sync_copy(maskv, mk_o.at[pl.ds(base, SPT)])
    pltpu.sync_copy(qcv, qc_o.at[wid])

    for cp in dcps:
        cp.wait()
    pltpu.sync_copy(dsv, distr_o.at[pl.ds(base, SPT)])
    pltpu.sync_copy(ngv, neigh_o.at[:, pl.ds(base, SPT)])


@functools.cache
def _build_sc_gather():
    return pl.kernel(
        _sc_gather_body,
        out_type=[
            jax.ShapeDtypeStruct((NP, D), jnp.float32),
            jax.ShapeDtypeStruct((K, NP, D), jnp.float32),
            jax.ShapeDtypeStruct((NTILES, 16, SPT), jnp.float32),
            jax.ShapeDtypeStruct((NP,), jnp.float32),
            jax.ShapeDtypeStruct((NP,), jnp.int32),
            jax.ShapeDtypeStruct((NP,), jnp.int32),
            jax.ShapeDtypeStruct((NP,), jnp.int32),
        ],
        mesh=_sc_mesh(),
        scratch_types=[
            pltpu.VMEM((SPT,), jnp.int32),
            pltpu.VMEM((SPT,), jnp.int32),
            pltpu.VMEM((SPT,), jnp.float32),
            pltpu.VMEM((SPT,), jnp.float32),
            pltpu.VMEM((SPT,), jnp.float32),
            pltpu.VMEM((SPT,), jnp.int32),
            pltpu.VMEM((SPT,), jnp.int32),
            pltpu.VMEM((SPT,), jnp.int32),
            pltpu.VMEM((16, SPT), jnp.float32),
            pltpu.VMEM((SPT, D), jnp.float32),
            pltpu.VMEM((K, SPT, D), jnp.float32),
            pltpu.SemaphoreType.DMA,
            pltpu.SemaphoreType.DMA,
        ],
    )


def _score_kernel(sdes_ref, neigh_ref, distr_ref, qc_ref, q1c_ref,
                  xxc_ref, yyc_ref, drow_ref, dcol_ref,
                  mm_ref, mx_ref, qlt_ref):
    sdes = sdes_ref[...]
    mx = jnp.full((NP, 1), -jnp.inf, jnp.float32)
    qsel = jnp.zeros((NP, 1), jnp.float32)
    for k in range(K):
        s = jnp.sum(sdes * neigh_ref[k], axis=-1, keepdims=True)
        better = s > mx
        mx = jnp.where(better, s, mx)
        qsel = jnp.where(better, qc_ref[:, k:k + 1], qsel)
    mx_ref[...] = mx
    qlt_ref[...] = (q1c_ref[...] + qsel) * 0.5

    mm = lax.dot_general(sdes, distr_ref[...],
                         (((1,), (1,)), ((), ())),
                         preferred_element_type=jnp.float32)
    dx = drow_ref[...] - xxc_ref[...]
    dy = dcol_ref[...] - yyc_ref[...]
    dis2 = dx * dx + dy * dy
    mm_ref[...] = jnp.where(dis2 < POS_R ** 2, 0.0, mm)


def _run_score(sdes, neigh, distr, qc, q1c, xxc, yyc, drowr, dcolr):
    return pl.pallas_call(
        _score_kernel,
        out_shape=[
            jax.ShapeDtypeStruct((NP, NP), jnp.float32),
            jax.ShapeDtypeStruct((NP, 1), jnp.float32),
            jax.ShapeDtypeStruct((NP, 1), jnp.float32),
        ],
    )(sdes, neigh, distr, qc, q1c, xxc, yyc, drowr, dcolr)


def _unshuffle(det):
    c = det[0, 0, BORDER:H - BORDER, BORDER:W - BORDER]
    c = c.reshape(NC, CELL, NC, CELL).transpose(1, 3, 0, 2).reshape(CELL * CELL, N)
    return jnp.pad(c, ((0, 0), (0, NP - N)), constant_values=-1.0)


def kernel(des1, det1, qlt1, des2, det2, qlt2, aflow):
    d1r = _unshuffle(det1)
    d2r = _unshuffle(det2)
    p1, pd, drow, dcol, tg1 = _run_argmax(d1r, d2r)

    tgt3 = jnp.pad(tg1[0, :N].reshape(NC, NC).T, ((0, 0), (0, 2)),
                   constant_values=-1).reshape(NC, 1, 32)
    sg = _run_gather1(des1.reshape(D, H, W), tgt3)
    sdes = jnp.pad(sg[:, :NC, :].transpose(1, 0, 2).reshape(N, D),
                   ((0, NP - N), (0, 0)))

    des2t = des2.reshape(D, HW).T

    distr, neigh, qc, q1, xx, yy, mk = _build_sc_gather()(
        p1.reshape(NP), pd.reshape(NP), des2t,
        aflow.reshape(-1), qlt1.reshape(-1), qlt2.reshape(-1))

    qcn = qc.transpose(0, 2, 1).reshape(NP, 16)
    mm, mxc, qltc = _run_score(
        sdes, neigh, distr, qcn, q1.reshape(NP, 1),
        xx.reshape(NP, 1), yy.reshape(NP, 1),
        drow.reshape(1, NP), dcol.reshape(1, NP))

    scores = jnp.concatenate([mxc[:N], mm[:N, :N]], axis=1)
    labels = jnp.zeros((N, N + 1), dtype=bool).at[:, :1].set(True)
    mask = (mk[:N] != 0).reshape(1, N)
    qlt = qltc[:N]
    return scores, labels, mask, qlt
